# Initial kernel scaffold; baseline (speedup 1.0000x reference)
import jax
import jax.numpy as jnp
from jax.experimental import pallas as pl


def kernel(x, enc_w1, enc_b1, enc_w2, enc_b2, cb_w, dec_w1, dec_b1, dec_w2, dec_b2):
    raise NotImplementedError("write your pallas kernel here")


# speedup vs baseline: 34.0627x; 34.0627x over previous
import jax
import jax.numpy as jnp
from jax.experimental import pallas as pl
from jax.experimental.pallas import tpu as pltpu

_K = 8
_V = 4
_TB = 256

_NEG = -3.0e38


def _kth_largest(l):
    t = jnp.max(l, axis=-1, keepdims=True)
    for _ in range(_K - 1):
        t = jnp.max(jnp.where(l < t, l, _NEG), axis=-1, keepdims=True)
    return t


def _body(x_ref, w1_ref, b1_ref, w2_ref, b2_ref, cb_ref,
          dw1_ref, db1_ref, dw2_ref, db2_ref, rec_ref, khot_ref):
    x = x_ref[...]
    khot = None
    for v in range(_V):
        h = jnp.dot(x, w1_ref[v], preferred_element_type=jnp.float32)
        h = jnp.maximum(h + b1_ref[v][None, :], 0.0)
        l = jnp.dot(h, w2_ref[v], preferred_element_type=jnp.float32)
        l = l + b2_ref[v][None, :]
        sel = (l >= _kth_largest(l)).astype(jnp.float32)
        khot = sel if khot is None else jnp.maximum(khot, sel)
    khot_ref[...] = khot
    q = jnp.dot(khot, cb_ref[...], preferred_element_type=jnp.float32)
    d = jnp.maximum(
        jnp.dot(q, dw1_ref[...], preferred_element_type=jnp.float32)
        + db1_ref[...][None, :], 0.0)
    rec_ref[...] = (
        jnp.dot(d, dw2_ref[...], preferred_element_type=jnp.float32)
        + db2_ref[...][None, :])


def kernel(x, enc_w1, enc_b1, enc_w2, enc_b2, cb_w,
           dec_w1, dec_b1, dec_w2, dec_b2):
    B, input_dim = x.shape
    V, _, n_hdim = enc_w1.shape
    Q = enc_w2.shape[2]
    n_embd = cb_w.shape[1]
    grid = (B // _TB,)

    full = lambda shape: pl.BlockSpec(shape, lambda i: (0,) * len(shape))
    rec, khot = pl.pallas_call(
        _body,
        grid=grid,
        in_specs=[
            pl.BlockSpec((_TB, input_dim), lambda i: (i, 0)),
            full((V, input_dim, n_hdim)),
            full((V, n_hdim)),
            full((V, n_hdim, Q)),
            full((V, Q)),
            full((Q, n_embd)),
            full((n_embd, n_hdim)),
            full((n_hdim,)),
            full((n_hdim, input_dim)),
            full((input_dim,)),
        ],
        out_specs=[
            pl.BlockSpec((_TB, input_dim), lambda i: (i, 0)),
            pl.BlockSpec((_TB, Q), lambda i: (i, 0)),
        ],
        out_shape=[
            jax.ShapeDtypeStruct((B, input_dim), jnp.float32),
            jax.ShapeDtypeStruct((B, Q), jnp.float32),
        ],
    )(x, enc_w1, enc_b1, enc_w2, enc_b2, cb_w, dec_w1, dec_b1, dec_w2, dec_b2)
    return (rec, khot, 0.0)
